# baseline (speedup 1.0000x reference)
import functools

import jax
import jax.experimental.shard_map
import jax.numpy as jnp
import numpy as np
from jax import lax
from jax.experimental import pallas as pl
from jax.experimental.pallas import tpu as pltpu
from jax.experimental.pallas import tpu_sc as plsc
from jax.sharding import Mesh, PartitionSpec as P

_N = 16384
_M = _N // 4
_K = 16
_F = 64
_RB = 64


def _fps_kernel(pts_s_ref, pts_ref, px_ref, py_ref, pz_ref, qp_ref):
    px = px_ref[:, :]
    py = py_ref[:, :]
    pz = pz_ref[:, :]
    iota = (jax.lax.broadcasted_iota(jnp.int32, (128, 128), 0) * 128
            + jax.lax.broadcasted_iota(jnp.int32, (128, 128), 1))
    dists0 = jnp.full((128, 128), jnp.inf, dtype=jnp.float32)

    def body(i, carry):
        dists, idx = carry
        lx = pts_s_ref[0, idx]
        ly = pts_s_ref[1, idx]
        lz = pts_s_ref[2, idx]
        row = pts_ref[pl.ds(idx, 1), :]
        qp_ref[pl.ds(i - 1, 1), :] = row
        dx = px - lx
        dy = py - ly
        dz = pz - lz
        d = dx * dx + dz * dz
        d = d + dy * dy
        dists = jnp.minimum(dists, d)
        m1 = jnp.max(dists, axis=1, keepdims=True)
        m = jnp.max(m1)
        cand = jnp.where(dists == m, iota, jnp.int32(_N))
        c1 = jnp.min(cand, axis=1, keepdims=True)
        nxt = jnp.min(c1)
        return dists, nxt

    _, idx = jax.lax.fori_loop(1, _M, body, (dists0, jnp.int32(0)))
    row = pts_ref[pl.ds(idx, 1), :]
    qp_ref[pl.ds(_M - 1, 1), :] = row


def _topk_kernel(qp_ref, px_ref, py_ref, pz_ref, idx_ref, d_ref):
    qp = qp_ref[:, :]
    qx = qp[:, 0:1]
    qy = qp[:, 1:2]
    qz = qp[:, 2:3]
    px = px_ref[:, :]
    py = py_ref[:, :]
    pz = pz_ref[:, :]
    qq = qx * qx + qz * qz
    qq = qq + qy * qy
    pp = px * px + pz * pz
    pp = pp + py * py
    qxb = qx.astype(jnp.bfloat16).astype(jnp.float32)
    qyb = qy.astype(jnp.bfloat16).astype(jnp.float32)
    qzb = qz.astype(jnp.bfloat16).astype(jnp.float32)
    pxb = px.astype(jnp.bfloat16).astype(jnp.float32)
    pyb = py.astype(jnp.bfloat16).astype(jnp.float32)
    pzb = pz.astype(jnp.bfloat16).astype(jnp.float32)
    t = qxb * pxb + qyb * pyb
    t = t + qzb * pzb
    d_ref[:, :] = (qq - 2.0 * t) + pp
    iota = jax.lax.broadcasted_iota(jnp.int32, (_RB, _N), 1)
    cols = []
    for _ in range(_K):
        dcur = d_ref[:, :]
        m = jnp.min(dcur, axis=1, keepdims=True)
        cand = jnp.where(dcur == m, iota, jnp.int32(_N))
        sel = jnp.min(cand, axis=1, keepdims=True)
        cols.append(sel)
        d_ref[:, :] = jnp.where(iota == sel, jnp.inf, dcur)
    idx_ref[:, :] = jnp.concatenate(cols, axis=1)


_NW = 32
_CQ = 32
_CR = _CQ * _K


def _make_sc_pool(m):
    q_per_w = m // _NW
    n_chunks = q_per_w // _CQ

    def body(feats_hbm, idx_hbm, out_hbm, idx_v, rows_v, pooled_v, sem):
        wid = lax.axis_index("s") * 2 + lax.axis_index("c")
        qbase = wid * q_per_w

        def chunk(c, carry):
            q0 = qbase + c * _CQ
            pltpu.sync_copy(idx_hbm.at[pl.ds(q0 * _K, _CR)], idx_v)
            pltpu.async_copy(feats_hbm.at[idx_v], rows_v, sem).wait()

            def qbody(qi, carry2):
                r0 = qi * _K
                for g in range(_F // 16):
                    acc = rows_v[r0, pl.ds(g * 16, 16)]
                    for k in range(1, _K):
                        acc = jnp.maximum(
                            acc, rows_v[r0 + k, pl.ds(g * 16, 16)])
                    pooled_v[qi, pl.ds(g * 16, 16)] = acc
                return carry2

            lax.fori_loop(0, _CQ, qbody, 0)
            pltpu.sync_copy(pooled_v, out_hbm.at[pl.ds(q0, _CQ)])
            return carry

        lax.fori_loop(0, n_chunks, chunk, 0)

    return functools.partial(
        pl.kernel,
        mesh=plsc.VectorSubcoreMesh(core_axis_name="c", subcore_axis_name="s"),
        out_type=jax.ShapeDtypeStruct((m, _F), jnp.float32),
        scratch_types=[
            pltpu.VMEM((_CR,), jnp.int32),
            pltpu.VMEM((_CR, 128), jnp.float32),
            pltpu.VMEM((_CQ, _F), jnp.float32),
            pltpu.SemaphoreType.DMA,
        ],
    )(body)


_sc_pool_full = _make_sc_pool(_M)
_sc_pool_half = _make_sc_pool(_M // 2)


def _pipeline(x, pid, parts):
    mq = _M // parts
    pts = x[:, :3]
    feats = x[:, 3:]
    pts_pad = jnp.pad(pts, ((0, 0), (0, 5)))
    px = pts[:, 0].reshape(128, 128)
    py = pts[:, 1].reshape(128, 128)
    pz = pts[:, 2].reshape(128, 128)
    px_l = pts[:, 0].reshape(1, _N)
    py_l = pts[:, 1].reshape(1, _N)
    pz_l = pts[:, 2].reshape(1, _N)

    pts_s = jnp.transpose(pts)
    qp = pl.pallas_call(
        _fps_kernel,
        in_specs=[
            pl.BlockSpec(memory_space=pltpu.SMEM),
            pl.BlockSpec(memory_space=pltpu.VMEM),
            pl.BlockSpec(memory_space=pltpu.VMEM),
            pl.BlockSpec(memory_space=pltpu.VMEM),
            pl.BlockSpec(memory_space=pltpu.VMEM),
        ],
        out_shape=jax.ShapeDtypeStruct((_M, 8), jnp.float32),
    )(pts_s, pts_pad, px, py, pz)

    qp_part = lax.dynamic_slice_in_dim(qp, pid * mq, mq)

    nn_idx = pl.pallas_call(
        _topk_kernel,
        grid=(mq // _RB,),
        in_specs=[
            pl.BlockSpec((_RB, 8), lambda i: (i, 0)),
            pl.BlockSpec((1, _N), lambda i: (0, 0)),
            pl.BlockSpec((1, _N), lambda i: (0, 0)),
            pl.BlockSpec((1, _N), lambda i: (0, 0)),
        ],
        out_specs=pl.BlockSpec((_RB, _K), lambda i: (i, 0)),
        out_shape=jax.ShapeDtypeStruct((mq, _K), jnp.int32),
        scratch_shapes=[pltpu.VMEM((_RB, _N), jnp.float32)],
    )(qp_part, px_l, py_l, pz_l)

    feats_pad = jnp.pad(feats, ((0, 0), (0, 128 - _F)))
    sc_pool = _sc_pool_full if parts == 1 else _sc_pool_half
    pooled = sc_pool(feats_pad, nn_idx.reshape(mq * _K))

    return jnp.concatenate([qp_part[:, :3], pooled], axis=1)


def kernel(x):
    devs = jax.devices()
    tpu_devs = [d for d in devs if d.platform == "tpu"]
    if len(tpu_devs) >= 2:
        mesh = Mesh(np.asarray(tpu_devs[:2]), ("tc",))
        f = jax.experimental.shard_map.shard_map(
            lambda xs: _pipeline(xs, lax.axis_index("tc"), 2),
            mesh=mesh, in_specs=P(), out_specs=P("tc", None),
            check_rep=False)
        return f(x)
    return _pipeline(x, 0, 1)


# speedup vs baseline: 1.1936x; 1.1936x over previous
import functools

import jax
import jax.experimental.shard_map
import jax.numpy as jnp
import numpy as np
from jax import lax
from jax.experimental import pallas as pl
from jax.experimental.pallas import tpu as pltpu
from jax.experimental.pallas import tpu_sc as plsc
from jax.sharding import Mesh, PartitionSpec as P

_N = 16384
_M = _N // 4
_K = 16
_F = 64
_RB = 64


def _fps_kernel(pts_s_ref, pts_ref, px_ref, py_ref, pz_ref, qp_ref):
    px = px_ref[:, :]
    py = py_ref[:, :]
    pz = pz_ref[:, :]
    iota = (jax.lax.broadcasted_iota(jnp.int32, (8, 2048), 0) * 2048
            + jax.lax.broadcasted_iota(jnp.int32, (8, 2048), 1)
            ).astype(jnp.float32)
    dists0 = jnp.full((8, 2048), jnp.inf, dtype=jnp.float32)

    def body(i, carry):
        dists, idx = carry
        lx = pts_s_ref[0, idx]
        ly = pts_s_ref[1, idx]
        lz = pts_s_ref[2, idx]
        row = pts_ref[pl.ds(idx, 1), :]
        qp_ref[pl.ds(i - 1, 1), :] = row
        dx = px - lx
        dy = py - ly
        dz = pz - lz
        d = dx * dx + dz * dz
        d = d + dy * dy
        dists = jnp.minimum(dists, d)
        m1 = jnp.max(dists, axis=1, keepdims=True)
        m = jnp.max(m1)
        cand = jnp.where(dists == m, iota, jnp.float32(_N))
        c1 = jnp.min(cand, axis=1, keepdims=True)
        nxt = jnp.min(c1).astype(jnp.int32)
        return dists, nxt

    _, idx = jax.lax.fori_loop(1, _M, body, (dists0, jnp.int32(0)))
    row = pts_ref[pl.ds(idx, 1), :]
    qp_ref[pl.ds(_M - 1, 1), :] = row


def _topk_kernel(qp_ref, px_ref, py_ref, pz_ref, idx_ref, d_ref):
    qp = qp_ref[:, :]
    qx = qp[:, 0:1]
    qy = qp[:, 1:2]
    qz = qp[:, 2:3]
    px = px_ref[:, :]
    py = py_ref[:, :]
    pz = pz_ref[:, :]
    qq = qx * qx + qz * qz
    qq = qq + qy * qy
    pp = px * px + pz * pz
    pp = pp + py * py
    qxb = qx.astype(jnp.bfloat16).astype(jnp.float32)
    qyb = qy.astype(jnp.bfloat16).astype(jnp.float32)
    qzb = qz.astype(jnp.bfloat16).astype(jnp.float32)
    pxb = px.astype(jnp.bfloat16).astype(jnp.float32)
    pyb = py.astype(jnp.bfloat16).astype(jnp.float32)
    pzb = pz.astype(jnp.bfloat16).astype(jnp.float32)
    t = qxb * pxb + qyb * pyb
    t = t + qzb * pzb
    d_ref[:, :] = (qq - 2.0 * t) + pp
    iota = jax.lax.broadcasted_iota(jnp.int32, (_RB, _N), 1)
    cols = []
    for _ in range(_K):
        dcur = d_ref[:, :]
        m = jnp.min(dcur, axis=1, keepdims=True)
        cand = jnp.where(dcur == m, iota, jnp.int32(_N))
        sel = jnp.min(cand, axis=1, keepdims=True)
        cols.append(sel)
        d_ref[:, :] = jnp.where(iota == sel, jnp.inf, dcur)
    idx_ref[:, :] = jnp.concatenate(cols, axis=1)


_NW = 32
_CQ = 32
_CR = _CQ * _K


def _make_sc_pool(m):
    q_per_w = m // _NW
    n_chunks = q_per_w // _CQ

    def body(feats_hbm, idx_hbm, out_hbm, idx_v, rows_v, pooled_v, sem):
        wid = lax.axis_index("s") * 2 + lax.axis_index("c")
        qbase = wid * q_per_w

        def chunk(c, carry):
            q0 = qbase + c * _CQ
            pltpu.sync_copy(idx_hbm.at[pl.ds(q0 * _K, _CR)], idx_v)
            pltpu.async_copy(feats_hbm.at[idx_v], rows_v, sem).wait()

            def qbody(qi, carry2):
                r0 = qi * _K
                for g in range(_F // 16):
                    acc = rows_v[r0, pl.ds(g * 16, 16)]
                    for k in range(1, _K):
                        acc = jnp.maximum(
                            acc, rows_v[r0 + k, pl.ds(g * 16, 16)])
                    pooled_v[qi, pl.ds(g * 16, 16)] = acc
                return carry2

            lax.fori_loop(0, _CQ, qbody, 0)
            pltpu.sync_copy(pooled_v, out_hbm.at[pl.ds(q0, _CQ)])
            return carry

        lax.fori_loop(0, n_chunks, chunk, 0)

    return functools.partial(
        pl.kernel,
        mesh=plsc.VectorSubcoreMesh(core_axis_name="c", subcore_axis_name="s"),
        out_type=jax.ShapeDtypeStruct((m, _F), jnp.float32),
        scratch_types=[
            pltpu.VMEM((_CR,), jnp.int32),
            pltpu.VMEM((_CR, 128), jnp.float32),
            pltpu.VMEM((_CQ, _F), jnp.float32),
            pltpu.SemaphoreType.DMA,
        ],
    )(body)


_sc_pool_full = _make_sc_pool(_M)
_sc_pool_half = _make_sc_pool(_M // 2)


def _pipeline(x, pid, parts):
    mq = _M // parts
    pts = x[:, :3]
    feats = x[:, 3:]
    pts_pad = jnp.pad(pts, ((0, 0), (0, 5)))
    px = pts[:, 0].reshape(8, 2048)
    py = pts[:, 1].reshape(8, 2048)
    pz = pts[:, 2].reshape(8, 2048)
    px_l = pts[:, 0].reshape(1, _N)
    py_l = pts[:, 1].reshape(1, _N)
    pz_l = pts[:, 2].reshape(1, _N)

    pts_s = jnp.transpose(pts)
    qp = pl.pallas_call(
        _fps_kernel,
        in_specs=[
            pl.BlockSpec(memory_space=pltpu.SMEM),
            pl.BlockSpec(memory_space=pltpu.VMEM),
            pl.BlockSpec(memory_space=pltpu.VMEM),
            pl.BlockSpec(memory_space=pltpu.VMEM),
            pl.BlockSpec(memory_space=pltpu.VMEM),
        ],
        out_shape=jax.ShapeDtypeStruct((_M, 8), jnp.float32),
    )(pts_s, pts_pad, px, py, pz)

    qp_part = lax.dynamic_slice_in_dim(qp, pid * mq, mq)

    nn_idx = pl.pallas_call(
        _topk_kernel,
        grid=(mq // _RB,),
        in_specs=[
            pl.BlockSpec((_RB, 8), lambda i: (i, 0)),
            pl.BlockSpec((1, _N), lambda i: (0, 0)),
            pl.BlockSpec((1, _N), lambda i: (0, 0)),
            pl.BlockSpec((1, _N), lambda i: (0, 0)),
        ],
        out_specs=pl.BlockSpec((_RB, _K), lambda i: (i, 0)),
        out_shape=jax.ShapeDtypeStruct((mq, _K), jnp.int32),
        scratch_shapes=[pltpu.VMEM((_RB, _N), jnp.float32)],
    )(qp_part, px_l, py_l, pz_l)

    feats_pad = jnp.pad(feats, ((0, 0), (0, 128 - _F)))
    sc_pool = _sc_pool_full if parts == 1 else _sc_pool_half
    pooled = sc_pool(feats_pad, nn_idx.reshape(mq * _K))

    return jnp.concatenate([qp_part[:, :3], pooled], axis=1)


def kernel(x):
    devs = jax.devices()
    tpu_devs = [d for d in devs if d.platform == "tpu"]
    if len(tpu_devs) >= 2:
        mesh = Mesh(np.asarray(tpu_devs[:2]), ("tc",))
        f = jax.experimental.shard_map.shard_map(
            lambda xs: _pipeline(xs, lax.axis_index("tc"), 2),
            mesh=mesh, in_specs=P(), out_specs=P("tc", None),
            check_rep=False)
        return f(x)
    return _pipeline(x, 0, 1)


# speedup vs baseline: 1.2573x; 1.0534x over previous
import functools

import jax
import jax.experimental.shard_map
import jax.numpy as jnp
import numpy as np
from jax import lax
from jax.experimental import pallas as pl
from jax.experimental.pallas import tpu as pltpu
from jax.experimental.pallas import tpu_sc as plsc
from jax.sharding import Mesh, PartitionSpec as P

_N = 16384
_M = _N // 4
_K = 16
_F = 64
_RB = 64


def _fps_kernel(pts_s_ref, pts_ref, px_ref, py_ref, pz_ref, qp_ref):
    px = px_ref[:, :]
    py = py_ref[:, :]
    pz = pz_ref[:, :]
    iota = (jax.lax.broadcasted_iota(jnp.int32, (8, 2048), 0) * 2048
            + jax.lax.broadcasted_iota(jnp.int32, (8, 2048), 1)
            ).astype(jnp.float32)
    dists0 = jnp.full((8, 2048), jnp.inf, dtype=jnp.float32)

    def body(i, carry):
        dists, idx = carry
        lx = pts_s_ref[0, idx]
        ly = pts_s_ref[1, idx]
        lz = pts_s_ref[2, idx]
        row = pts_ref[pl.ds(idx, 1), :]
        qp_ref[pl.ds(i - 1, 1), :] = row
        dx = px - lx
        dy = py - ly
        dz = pz - lz
        d = dx * dx + dz * dz
        d = d + dy * dy
        dists = jnp.minimum(dists, d)
        m1 = jnp.max(dists, axis=1, keepdims=True)
        m = jnp.max(m1)
        cand = jnp.where(dists == m, iota, jnp.float32(_N))
        c1 = jnp.min(cand, axis=1, keepdims=True)
        nxt = jnp.min(c1).astype(jnp.int32)
        return dists, nxt

    _, idx = jax.lax.fori_loop(1, _M, body, (dists0, jnp.int32(0)))
    row = pts_ref[pl.ds(idx, 1), :]
    qp_ref[pl.ds(_M - 1, 1), :] = row


def _topk_kernel(qp_ref, px_ref, py_ref, pz_ref, idx_ref, d_ref):
    qp = qp_ref[:, :]
    qx = qp[:, 0:1]
    qy = qp[:, 1:2]
    qz = qp[:, 2:3]
    px = px_ref[:, :]
    py = py_ref[:, :]
    pz = pz_ref[:, :]
    qq = qx * qx + qz * qz
    qq = qq + qy * qy
    pp = px * px + pz * pz
    pp = pp + py * py
    qxb = qx.astype(jnp.bfloat16).astype(jnp.float32)
    qyb = qy.astype(jnp.bfloat16).astype(jnp.float32)
    qzb = qz.astype(jnp.bfloat16).astype(jnp.float32)
    pxb = px.astype(jnp.bfloat16).astype(jnp.float32)
    pyb = py.astype(jnp.bfloat16).astype(jnp.float32)
    pzb = pz.astype(jnp.bfloat16).astype(jnp.float32)
    t = qxb * pxb + qyb * pyb
    t = t + qzb * pzb
    d_ref[:, :] = (qq - 2.0 * t) + pp
    iota = jax.lax.broadcasted_iota(
        jnp.int32, (_RB, _N), 1).astype(jnp.float32)
    cols = []
    for _ in range(_K):
        dcur = d_ref[:, :]
        m = jnp.min(dcur, axis=1, keepdims=True)
        cand = jnp.where(dcur == m, iota, jnp.float32(_N))
        sel = jnp.min(cand, axis=1, keepdims=True)
        cols.append(sel)
        d_ref[:, :] = jnp.where(iota == sel, jnp.inf, dcur)
    idx_ref[:, :] = jnp.concatenate(cols, axis=1).astype(jnp.int32)


_NW = 32
_CQ = 32
_CR = _CQ * _K


def _make_sc_pool(m):
    q_per_w = m // _NW
    n_chunks = q_per_w // _CQ

    def body(feats_hbm, idx_hbm, out_hbm, idx_v, rows_v, pooled_v, sem):
        wid = lax.axis_index("s") * 2 + lax.axis_index("c")
        qbase = wid * q_per_w

        def chunk(c, carry):
            q0 = qbase + c * _CQ
            pltpu.sync_copy(idx_hbm.at[pl.ds(q0 * _K, _CR)], idx_v)
            pltpu.async_copy(feats_hbm.at[idx_v], rows_v, sem).wait()

            def qbody(qi, carry2):
                r0 = qi * _K
                for g in range(_F // 16):
                    acc = rows_v[r0, pl.ds(g * 16, 16)]
                    for k in range(1, _K):
                        acc = jnp.maximum(
                            acc, rows_v[r0 + k, pl.ds(g * 16, 16)])
                    pooled_v[qi, pl.ds(g * 16, 16)] = acc
                return carry2

            lax.fori_loop(0, _CQ, qbody, 0)
            pltpu.sync_copy(pooled_v, out_hbm.at[pl.ds(q0, _CQ)])
            return carry

        lax.fori_loop(0, n_chunks, chunk, 0)

    return functools.partial(
        pl.kernel,
        mesh=plsc.VectorSubcoreMesh(core_axis_name="c", subcore_axis_name="s"),
        out_type=jax.ShapeDtypeStruct((m, _F), jnp.float32),
        scratch_types=[
            pltpu.VMEM((_CR,), jnp.int32),
            pltpu.VMEM((_CR, 128), jnp.float32),
            pltpu.VMEM((_CQ, _F), jnp.float32),
            pltpu.SemaphoreType.DMA,
        ],
    )(body)


_sc_pool_full = _make_sc_pool(_M)
_sc_pool_half = _make_sc_pool(_M // 2)


def _pipeline(x, pid, parts):
    mq = _M // parts
    pts = x[:, :3]
    feats = x[:, 3:]
    pts_pad = jnp.pad(pts, ((0, 0), (0, 5)))
    px = pts[:, 0].reshape(8, 2048)
    py = pts[:, 1].reshape(8, 2048)
    pz = pts[:, 2].reshape(8, 2048)
    px_l = pts[:, 0].reshape(1, _N)
    py_l = pts[:, 1].reshape(1, _N)
    pz_l = pts[:, 2].reshape(1, _N)

    pts_s = jnp.transpose(pts)
    qp = pl.pallas_call(
        _fps_kernel,
        in_specs=[
            pl.BlockSpec(memory_space=pltpu.SMEM),
            pl.BlockSpec(memory_space=pltpu.VMEM),
            pl.BlockSpec(memory_space=pltpu.VMEM),
            pl.BlockSpec(memory_space=pltpu.VMEM),
            pl.BlockSpec(memory_space=pltpu.VMEM),
        ],
        out_shape=jax.ShapeDtypeStruct((_M, 8), jnp.float32),
    )(pts_s, pts_pad, px, py, pz)

    qp_part = lax.dynamic_slice_in_dim(qp, pid * mq, mq)

    nn_idx = pl.pallas_call(
        _topk_kernel,
        grid=(mq // _RB,),
        in_specs=[
            pl.BlockSpec((_RB, 8), lambda i: (i, 0)),
            pl.BlockSpec((1, _N), lambda i: (0, 0)),
            pl.BlockSpec((1, _N), lambda i: (0, 0)),
            pl.BlockSpec((1, _N), lambda i: (0, 0)),
        ],
        out_specs=pl.BlockSpec((_RB, _K), lambda i: (i, 0)),
        out_shape=jax.ShapeDtypeStruct((mq, _K), jnp.int32),
        scratch_shapes=[pltpu.VMEM((_RB, _N), jnp.float32)],
    )(qp_part, px_l, py_l, pz_l)

    feats_pad = jnp.pad(feats, ((0, 0), (0, 128 - _F)))
    sc_pool = _sc_pool_full if parts == 1 else _sc_pool_half
    pooled = sc_pool(feats_pad, nn_idx.reshape(mq * _K))

    return jnp.concatenate([qp_part[:, :3], pooled], axis=1)


def kernel(x):
    devs = jax.devices()
    tpu_devs = [d for d in devs if d.platform == "tpu"]
    if len(tpu_devs) >= 2:
        mesh = Mesh(np.asarray(tpu_devs[:2]), ("tc",))
        f = jax.experimental.shard_map.shard_map(
            lambda xs: _pipeline(xs, lax.axis_index("tc"), 2),
            mesh=mesh, in_specs=P(), out_specs=P("tc", None),
            check_rep=False)
        return f(x)
    return _pipeline(x, 0, 1)
